# Initial kernel scaffold; baseline (speedup 1.0000x reference)
import jax
import jax.numpy as jnp
from jax.experimental import pallas as pl


def kernel(indices, values):
    raise NotImplementedError("write your pallas kernel here")


# speedup vs baseline: 3.7718x; 3.7718x over previous
import functools

import jax
import jax.numpy as jnp
from jax import lax
from jax.experimental import pallas as pl
from jax.experimental.pallas import tpu as pltpu
from jax.experimental.pallas import tpu_sc as plsc


def kernel(indices, values):
    B, H = indices.shape
    V, D = values.shape
    N = B * H

    info = plsc.get_sparse_core_info()
    NC, NS = info.num_cores, info.num_subcores
    NW = NC * NS
    n_per_w = N // NW
    C = 128
    n_chunks = n_per_w // C

    idx_flat = indices.reshape(N).astype(jnp.int32)

    @functools.partial(
        pl.kernel,
        mesh=plsc.VectorSubcoreMesh(core_axis_name="c", subcore_axis_name="s"),
        out_type=jax.ShapeDtypeStruct((N, D), jnp.float32),
        scratch_types=[
            pltpu.VMEM((C,), jnp.int32),
            pltpu.VMEM((C, D), jnp.float32),
            pltpu.SemaphoreType.DMA,
        ],
        compiler_params=pltpu.CompilerParams(use_tc_tiling_on_sc=False),
    )
    def gather_kernel(table_hbm, idx_hbm, out_hbm, idx_v, rows_v, sem):
        wid = lax.axis_index("s") * NC + lax.axis_index("c")
        base = wid * n_per_w

        def chunk(i, carry):
            off = base + i * C
            pltpu.sync_copy(idx_hbm.at[pl.ds(off, C)], idx_v)
            pltpu.async_copy(table_hbm.at[idx_v], rows_v, sem).wait()
            pltpu.sync_copy(rows_v, out_hbm.at[pl.ds(off, C)])
            return carry

        lax.fori_loop(0, n_chunks, chunk, 0)

    out = gather_kernel(values, idx_flat)
    return out.reshape(B, H, D)


# speedup vs baseline: 4.6779x; 1.2402x over previous
import functools

import jax
import jax.numpy as jnp
from jax import lax
from jax.experimental import pallas as pl
from jax.experimental.pallas import tpu as pltpu
from jax.experimental.pallas import tpu_sc as plsc


def kernel(indices, values):
    B, H = indices.shape
    V, D = values.shape
    N = B * H

    info = plsc.get_sparse_core_info()
    NC, NS = info.num_cores, info.num_subcores
    NW = NC * NS
    n_per_w = N // NW
    C = 128
    n_chunks = n_per_w // C
    NBUF = 10
    n_outer = n_chunks // NBUF

    idx3 = indices.reshape(NW, n_chunks, C).astype(jnp.int32)

    @functools.partial(
        pl.kernel,
        mesh=plsc.VectorSubcoreMesh(core_axis_name="c", subcore_axis_name="s"),
        out_type=jax.ShapeDtypeStruct((N, D), jnp.float32),
        scratch_types=[
            pltpu.VMEM((n_chunks, C), jnp.int32),
            pltpu.VMEM((NBUF, C, D), jnp.float32),
        ]
        + [pltpu.SemaphoreType.DMA] * (2 * NBUF),
        compiler_params=pltpu.CompilerParams(use_tc_tiling_on_sc=False),
    )
    def gather_kernel(table_hbm, idx_hbm, out_hbm, idx_v, rows_v, *sems):
        gsem = sems[:NBUF]
        wsem = sems[NBUF:]
        wid = lax.axis_index("s") * NC + lax.axis_index("c")
        base = wid * n_per_w

        def gather_start(i, k):
            pltpu.async_copy(table_hbm.at[idx_v.at[i]], rows_v.at[k], gsem[k])

        def gather_wait(i, k):
            pltpu.make_async_copy(
                table_hbm.at[idx_v.at[i]], rows_v.at[k], gsem[k]
            ).wait()

        def write_start(i, k):
            pltpu.async_copy(
                rows_v.at[k], out_hbm.at[pl.ds(base + i * C, C)], wsem[k]
            )

        def write_wait(k):
            pltpu.make_async_copy(
                rows_v.at[k], out_hbm.at[pl.ds(base, C)], wsem[k]
            ).wait()

        pltpu.sync_copy(idx_hbm.at[wid], idx_v)

        for k in range(NBUF):
            gather_start(k, k)

        def outer(o, carry):
            for k in range(NBUF):
                i = o * NBUF + k
                gather_wait(i, k)
                write_start(i, k)
                write_wait(k)
                gather_start(i + NBUF, k)
            return carry

        lax.fori_loop(0, n_outer - 1, outer, 0)

        for k in range(NBUF):
            i = (n_outer - 1) * NBUF + k
            gather_wait(i, k)
            write_start(i, k)
        for k in range(NBUF):
            write_wait(k)

    out = gather_kernel(values, idx3)
    return out.reshape(B, H, D)
